# baseline (speedup 1.0000x reference)
import functools

import jax
import jax.numpy as jnp
from jax import lax
from jax.experimental import pallas as pl
from jax.experimental.pallas import tpu as pltpu
from jax.experimental.pallas import tpu_sc as plsc

NUM_CARDS = 1000000
HIDDEN_DIM = 32
BATCH = 4096
HIST = 200

_info = plsc.get_sparse_core_info()
NC, NS, L = _info.num_cores, _info.num_subcores, _info.num_lanes
NW = NC * NS

NB = BATCH // NW
B_PER_W = NB * HIST
TCH = 4
CROWS = TCH * NB
NCH = HIST // TCH
NPAIR = NCH // 2
DG = HIDDEN_DIM // 8


def _body(cards_hbm, emb_hbm, out_hbm, idx_all, idxT, rows0, rows1,
          tb0, tb1, tb2, tb3, sg0, sg1, sw0, sw1, sw2, sw3):
    wid = lax.axis_index("s") * NC + lax.axis_index("c")
    base = wid * B_PER_W
    iota = lax.iota(jnp.int32, L)

    pltpu.sync_copy(cards_hbm.at[pl.ds(base, B_PER_W)], idx_all)

    def build(i, _):
        t = i // (NB // L)
        bg = i % (NB // L)
        src = (bg * L + iota) * HIST + t
        vals = plsc.load_gather(idx_all, [src]) + 1
        idxT[pl.ds(t * NB + bg * L, L)] = vals
        return 0

    lax.fori_loop(0, HIST * (NB // L), build, 0, unroll=8)

    def idx_slice(c):
        return idxT.at[pl.ds(pl.multiple_of(c * CROWS, CROWS), CROWS)]

    def start_gather(c, rows, sem):
        pltpu.async_copy(emb_hbm.at[idx_slice(c)], rows, sem)

    def wait_gather(c, rows, sem):
        pltpu.make_async_copy(emb_hbm.at[idx_slice(c)], rows, sem).wait()

    def start_write(t, tb, sem):
        for g in range(DG):
            pltpu.async_copy(
                tb.at[pl.ds(g * 1024, 1024)], out_hbm.at[t, g, wid], sem
            )

    def wait_write(t, tb, sem):
        for g in range(DG):
            pltpu.make_async_copy(
                tb.at[pl.ds(g * 1024, 1024)], out_hbm.at[t, g, wid], sem
            ).wait()

    tbufs = (tb0, tb1, tb2, tb3)
    swaps = (sw0, sw1, sw2, sw3)
    posbase = iota * NB

    def transpose_t(rows, tl, tb):
        rbase = tl * NB

        def row(j, _):
            r = rbase + j
            v0 = rows[r, pl.ds(0, L)]
            v1 = rows[r, pl.ds(L, L)]
            pos = posbase + j
            plsc.store_scatter(tb, [pos], v0)
            plsc.store_scatter(tb, [pos + L * NB], v1)
            return 0

        lax.fori_loop(0, NB, row, 0, unroll=8)

    def handle_chunk(c, rows, rows_nxt, sem, sem_nxt):
        @pl.when(c + 1 < NCH)
        def _():
            start_gather(c + 1, rows_nxt, sem_nxt)

        wait_gather(c, rows, sem)
        for tl in range(TCH):
            t = c * TCH + tl

            @pl.when(c >= 1)
            def _():
                wait_write(t - TCH, tbufs[tl], swaps[tl])

            transpose_t(rows, tl, tbufs[tl])
            start_write(t, tbufs[tl], swaps[tl])

    start_gather(0, rows0, sg0)

    def pair(g, _):
        c0 = 2 * g
        handle_chunk(c0, rows0, rows1, sg0, sg1)
        handle_chunk(c0 + 1, rows1, rows0, sg1, sg0)
        return 0

    lax.fori_loop(0, NPAIR, pair, 0)

    for tl in range(TCH):
        t = HIST - TCH + tl
        wait_write(t, tbufs[tl], (sw0, sw1, sw2, sw3)[tl])


TCOLS_FULL = 7812
JW = TCOLS_FULL // NW


def _tbody(embT_hbm, tail_hbm, emb4_hbm, tin0, tin1, tout0, tout1,
           si0, si1, so0, so1):
    wid = lax.axis_index("s") * NC + lax.axis_index("c")
    iota = lax.iota(jnp.int32, L)

    def col_of(jj):
        return jj * NW + wid

    def start_in(j, tin, sem):
        pltpu.async_copy(
            embT_hbm.at[pl.ds(0, HIDDEN_DIM), pl.ds(j * 128, 128)], tin, sem
        )

    def wait_in(j, tin, sem):
        pltpu.make_async_copy(
            embT_hbm.at[pl.ds(0, HIDDEN_DIM), pl.ds(j * 128, 128)], tin, sem
        ).wait()

    def start_out(j, tout, sem):
        pltpu.async_copy(tout, emb4_hbm.at[pl.ds(j * 4096, 4096)], sem)

    def wait_out(j, tout, sem):
        pltpu.make_async_copy(
            tout, emb4_hbm.at[pl.ds(j * 4096, 4096)], sem
        ).wait()

    def transpose_col(tin, tout):
        for c in range(128 // L):
            pv = iota * HIDDEN_DIM + c * (L * HIDDEN_DIM)

            def col(d, _):
                vals = tin[d, pl.ds(c * L, L)]
                plsc.store_scatter(tout, [pv + d], vals)
                return 0

            lax.fori_loop(0, HIDDEN_DIM, col, 0, unroll=8)

    tins = (tin0, tin1)
    touts = (tout0, tout1)
    sis = (si0, si1)
    sos = (so0, so1)

    def handle(jj, p):
        j = col_of(jj)

        @pl.when(jj + 1 <= JW)
        def _():
            nxt = col_of(jj + 1)

            @pl.when(nxt < TCOLS_FULL)
            def _():
                start_in(nxt, tins[1 - p], sis[1 - p])

        wait_in(j, tins[p], sis[p])

        @pl.when(jj >= 2)
        def _():
            wait_out(col_of(jj - 2), touts[p], sos[p])

        transpose_col(tins[p], touts[p])
        start_out(j, touts[p], sos[p])

    start_in(col_of(0), tin0, si0)

    def pair(g, _):
        handle(2 * g, 0)
        handle(2 * g + 1, 1)
        return 0

    lax.fori_loop(0, JW // 2, pair, 0)

    n_strag = TCOLS_FULL - JW * NW

    @pl.when(wid < n_strag)
    def _():
        handle(JW, 0)
        wait_out(col_of(JW), touts[0], sos[0])

    @pl.when(wid >= n_strag)
    def _():
        wait_out(col_of(JW - 2), touts[0], sos[0])

    wait_out(col_of(JW - 1), touts[1], sos[1])

    @pl.when(wid == NW - 1)
    def _():
        pltpu.sync_copy(tail_hbm, tout0.at[pl.ds(0, 2048)])
        pltpu.sync_copy(
            tout0.at[pl.ds(0, 2048)], emb4_hbm.at[pl.ds(249984 * 128, 2048)]
        )


@jax.jit
def _relayout(embT, tail):
    mesh = plsc.VectorSubcoreMesh(core_axis_name="c", subcore_axis_name="s")
    fn = pl.kernel(
        _tbody,
        out_type=jax.ShapeDtypeStruct((32000000,), jnp.float32),
        mesh=mesh,
        scratch_types=[
            pltpu.VMEM((HIDDEN_DIM, 128), jnp.float32),
            pltpu.VMEM((HIDDEN_DIM, 128), jnp.float32),
            pltpu.VMEM((32 * 128,), jnp.float32),
            pltpu.VMEM((32 * 128,), jnp.float32),
            pltpu.SemaphoreType.DMA,
            pltpu.SemaphoreType.DMA,
            pltpu.SemaphoreType.DMA,
            pltpu.SemaphoreType.DMA,
        ],
        compiler_params=pltpu.CompilerParams(
            use_tc_tiling_on_sc=True, needs_layout_passes=False
        ),
    )
    return fn(embT, tail)


@jax.jit
def _embed(cards_flat, emb):
    mesh = plsc.VectorSubcoreMesh(core_axis_name="c", subcore_axis_name="s")
    fn = pl.kernel(
        _body,
        out_type=jax.ShapeDtypeStruct((HIST, DG, NW, 8 * NB), jnp.float32),
        mesh=mesh,
        scratch_types=[
            pltpu.VMEM((B_PER_W,), jnp.int32),
            pltpu.VMEM((B_PER_W,), jnp.int32),
            pltpu.VMEM((CROWS, HIDDEN_DIM), jnp.float32),
            pltpu.VMEM((CROWS, HIDDEN_DIM), jnp.float32),
            pltpu.VMEM((HIDDEN_DIM * NB,), jnp.float32),
            pltpu.VMEM((HIDDEN_DIM * NB,), jnp.float32),
            pltpu.VMEM((HIDDEN_DIM * NB,), jnp.float32),
            pltpu.VMEM((HIDDEN_DIM * NB,), jnp.float32),
            pltpu.SemaphoreType.DMA,
            pltpu.SemaphoreType.DMA,
            pltpu.SemaphoreType.DMA,
            pltpu.SemaphoreType.DMA,
            pltpu.SemaphoreType.DMA,
            pltpu.SemaphoreType.DMA,
        ],
        compiler_params=pltpu.CompilerParams(
            use_tc_tiling_on_sc=False, needs_layout_passes=False
        ),
    )
    return fn(cards_flat, emb)


def kernel(cards, emb):
    cards_flat = cards.reshape(-1).astype(jnp.int32)
    embT = emb.T
    tail = emb[NUM_CARDS - 64:].reshape(-1)
    emb_lin = _relayout(embT, tail).reshape(NUM_CARDS, HIDDEN_DIM)
    out4 = _embed(cards_flat, emb_lin)
    out5 = out4.reshape(HIST, DG, NW, 8, NB)
    return out5.transpose(2, 4, 0, 1, 3).reshape(BATCH, HIST, HIDDEN_DIM)


# speedup vs baseline: 1.1448x; 1.1448x over previous
import functools

import jax
import jax.numpy as jnp
from jax import lax
from jax.experimental import pallas as pl
from jax.experimental.pallas import tpu as pltpu
from jax.experimental.pallas import tpu_sc as plsc

NUM_CARDS = 1000000
HIDDEN_DIM = 32
BATCH = 4096
HIST = 200

_info = plsc.get_sparse_core_info()
NC, NS, L = _info.num_cores, _info.num_subcores, _info.num_lanes
NW = NC * NS

B = BATCH * HIST
B_PER_W = B // NW
CHUNK = 1600
NCHUNK = B_PER_W // CHUNK
NPAIR = NCHUNK // 2


def _body(cards_hbm, emb_hbm, out_hbm, idx_all, rows0, rows1, sg0, sg1, so0, so1):
    wid = lax.axis_index("s") * NC + lax.axis_index("c")
    base = wid * B_PER_W
    pltpu.sync_copy(cards_hbm.at[pl.ds(base, B_PER_W)], idx_all)

    def add1_chunk(c):
        def add1(i, _):
            sl = pl.ds(c * CHUNK + i * L, L)
            idx_all[sl] = idx_all[sl] + 1
            return 0

        lax.fori_loop(0, CHUNK // L, add1, 0, unroll=4)

    def idx_slice(c):
        return idx_all.at[pl.ds(pl.multiple_of(c * CHUNK, CHUNK), CHUNK)]

    def start_gather(c, rows, sem):
        pltpu.async_copy(emb_hbm.at[idx_slice(c)], rows, sem)

    def wait_gather(c, rows, sem):
        pltpu.make_async_copy(emb_hbm.at[idx_slice(c)], rows, sem).wait()

    def start_write(c, rows, sem):
        b0 = (base + c * CHUNK) // HIST
        for k in range(CHUNK // HIST):
            pltpu.async_copy(
                rows.at[pl.ds(k * HIST, HIST)], out_hbm.at[b0 + k], sem
            )

    def wait_write(c, rows, sem):
        b0 = (base + c * CHUNK) // HIST
        for k in range(CHUNK // HIST):
            pltpu.make_async_copy(
                rows.at[pl.ds(k * HIST, HIST)], out_hbm.at[b0 + k], sem
            ).wait()

    add1_chunk(0)
    start_gather(0, rows0, sg0)
    add1_chunk(1)

    def pair(g, _):
        c0 = 2 * g
        c1 = c0 + 1

        @pl.when(g >= 1)
        def _():
            wait_write(c0 - 1, rows1, so1)

        start_gather(c1, rows1, sg1)

        @pl.when(g < NPAIR - 1)
        def _():
            add1_chunk(c0 + 2)

        wait_gather(c0, rows0, sg0)
        start_write(c0, rows0, so0)

        @pl.when(g < NPAIR - 1)
        def _():
            wait_write(c0, rows0, so0)
            start_gather(c1 + 1, rows0, sg0)
            add1_chunk(c1 + 2)

        wait_gather(c1, rows1, sg1)
        start_write(c1, rows1, so1)
        return 0

    lax.fori_loop(0, NPAIR, pair, 0)

    wait_write(NCHUNK - 2, rows0, so0)
    wait_write(NCHUNK - 1, rows1, so1)


@jax.jit
def _embed(cards_flat, emb):
    mesh = plsc.VectorSubcoreMesh(core_axis_name="c", subcore_axis_name="s")
    fn = pl.kernel(
        _body,
        out_type=jax.ShapeDtypeStruct((BATCH, HIST, HIDDEN_DIM), jnp.float32),
        mesh=mesh,
        scratch_types=[
            pltpu.VMEM((B_PER_W,), jnp.int32),
            pltpu.VMEM((CHUNK, HIDDEN_DIM), jnp.float32),
            pltpu.VMEM((CHUNK, HIDDEN_DIM), jnp.float32),
            pltpu.SemaphoreType.DMA,
            pltpu.SemaphoreType.DMA,
            pltpu.SemaphoreType.DMA,
            pltpu.SemaphoreType.DMA,
        ],
        compiler_params=pltpu.CompilerParams(use_tc_tiling_on_sc=False),
    )
    return fn(cards_flat, emb)


def kernel(cards, emb):
    cards_flat = cards.reshape(-1).astype(jnp.int32)
    return _embed(cards_flat, emb)
